# baseline (speedup 1.0000x reference)
import functools

import jax
import jax.numpy as jnp
from jax.experimental import pallas as pl
from jax.experimental.pallas import tpu as pltpu


def _token_select_kernel(x_ref, cls_ref, wf_ref, bf_ref, ws_ref, bs_ref,
                         out_ref, key_ref, idx_ref):
    x = x_ref[0]
    n = x.shape[0]

    half = jax.lax.dot_general(x, wf_ref[...], (((1,), (1,)), ((), ())),
                               preferred_element_type=jnp.float32)
    half = jnp.maximum(half + bf_ref[...], 0.0)
    cls_half = jax.lax.dot_general(cls_ref[0], wf_ref[...],
                                   (((1,), (1,)), ((), ())),
                                   preferred_element_type=jnp.float32)
    cls_half = cls_half + bf_ref[...]
    cat = jnp.concatenate(
        [jnp.broadcast_to(cls_half, half.shape), half], axis=1)
    scores = jax.lax.dot_general(cat, ws_ref[...], (((1,), (1,)), ((), ())),
                                 preferred_element_type=jnp.float32)
    scores = jnp.maximum(scores + bs_ref[...], 0.0)

    key_ref[...] = scores
    idx_ref[...] = jax.lax.broadcasted_iota(jnp.int32, scores.shape, 0)
    out_ref[0] = x

    pos = jax.lax.broadcasted_iota(jnp.int32, scores.shape, 0)

    def stage(_, jk):
        j, k = jk
        v = key_ref[...]
        ix = idx_ref[...]
        pld = out_ref[0]
        is_hi = (pos & j) != 0
        down = (pos & k) != 0
        back = jnp.int32(n) - j
        pv = jnp.where(is_hi, pltpu.roll(v, j, 0), pltpu.roll(v, back, 0))
        pix = jnp.where(is_hi, pltpu.roll(ix, j, 0), pltpu.roll(ix, back, 0))
        ppl = jnp.where(is_hi, pltpu.roll(pld, j, 0), pltpu.roll(pld, back, 0))
        takes = ((pv > v) | ((pv == v) & (pix < ix))) ^ is_hi ^ down
        key_ref[...] = jnp.where(takes, pv, v)
        idx_ref[...] = jnp.where(takes, pix, ix)
        out_ref[0] = jnp.where(takes, ppl, pld)
        last = j == 1
        k2 = jnp.where(last, k * 2, k)
        j2 = jnp.where(last, k, j // 2)
        return j2, k2

    n_stages = 0
    m = 1
    while (1 << m) <= n:
        n_stages += m
        m += 1
    jax.lax.fori_loop(0, n_stages, stage, (jnp.int32(1), jnp.int32(2)),
                      unroll=False)


def kernel(start_patch_token, cls_token, W_f, b_f, W_s, b_s, Temperature):
    B, N, D = start_patch_token.shape
    S = W_s.shape[0]
    grid = (B,)
    out = pl.pallas_call(
        _token_select_kernel,
        grid=grid,
        in_specs=[
            pl.BlockSpec((1, N, D), lambda b: (b, 0, 0)),
            pl.BlockSpec((1, 1, D), lambda b: (b, 0, 0)),
            pl.BlockSpec(W_f.shape, lambda b: (0, 0)),
            pl.BlockSpec((1, W_f.shape[0]), lambda b: (0, 0)),
            pl.BlockSpec(W_s.shape, lambda b: (0, 0)),
            pl.BlockSpec((1, S), lambda b: (0, 0)),
        ],
        out_specs=pl.BlockSpec((1, N, D), lambda b: (b, 0, 0)),
        out_shape=jax.ShapeDtypeStruct((B, N, D), jnp.float32),
        scratch_shapes=[
            pltpu.VMEM((N, S), jnp.float32),
            pltpu.VMEM((N, S), jnp.int32),
        ],
        compiler_params=pltpu.CompilerParams(
            dimension_semantics=("arbitrary",)),
    )(start_patch_token, cls_token.reshape(B, 1, D), W_f,
      b_f.reshape(1, -1), W_s, b_s.reshape(1, -1))
    return out


# speedup vs baseline: 6.1220x; 6.1220x over previous
import jax
import jax.numpy as jnp
from jax.experimental import pallas as pl
from jax.experimental.pallas import tpu as pltpu


def _swap_mask(vb, va, ib, ia):
    return (vb > va) | ((vb == va) & (ib < ia))


def _stage_big(v, ix, pld, j, k, n):
    m = n // (2 * j)
    s = v.shape[-1]

    def split(a):
        r = a.reshape(m, 2, j, s)
        return r[:, 0], r[:, 1]

    va, vb = split(v)
    ia, ib = split(ix)
    pa, pb = split(pld)
    swap = _swap_mask(vb, va, ib, ia)
    if k < n:
        o = jax.lax.broadcasted_iota(jnp.int32, (m, 1, 1), 0)
        down = ((o * (2 * j)) & k) != 0
        swap = swap ^ down

    def merge(a, b):
        na = jnp.where(swap, b, a)
        nb = jnp.where(swap, a, b)
        return jnp.concatenate([na[:, None], nb[:, None]], axis=1).reshape(n, s)

    return merge(va, vb), merge(ia, ib), merge(pa, pb)


def _stage_small(v, ix, pld, j, k, n, pos):
    is_hi = (pos & j) != 0
    dirm = is_hi ^ ((pos & k) != 0)

    def partner(a):
        return jnp.where(is_hi, pltpu.roll(a, j, 0), pltpu.roll(a, n - j, 0))

    pv = partner(v)
    pix = partner(ix)
    ppl = partner(pld)
    takes = _swap_mask(pv, v, pix, ix) ^ dirm
    return (jnp.where(takes, pv, v), jnp.where(takes, pix, ix),
            jnp.where(takes, ppl, pld))


def _token_select_kernel(x_ref, cls_ref, wf_ref, bf_ref, ws_ref, bs_ref,
                         out_ref, key_ref, idx_ref):
    x = x_ref[0]
    n = x.shape[0]

    half = jax.lax.dot_general(x, wf_ref[...], (((1,), (1,)), ((), ())),
                               preferred_element_type=jnp.float32)
    half = jnp.maximum(half + bf_ref[...], 0.0)
    cls_half = jax.lax.dot_general(cls_ref[0], wf_ref[...],
                                   (((1,), (1,)), ((), ())),
                                   preferred_element_type=jnp.float32)
    cls_half = cls_half + bf_ref[...]
    cat = jnp.concatenate(
        [jnp.broadcast_to(cls_half, half.shape), half], axis=1)
    scores = jax.lax.dot_general(cat, ws_ref[...], (((1,), (1,)), ((), ())),
                                 preferred_element_type=jnp.float32)
    scores = jnp.maximum(scores + bs_ref[...], 0.0)

    pos = jax.lax.broadcasted_iota(jnp.int32, scores.shape, 0)
    key_ref[...] = scores
    idx_ref[...] = pos
    out_ref[0] = x

    shifts = []
    jj = n // 2
    while jj >= 1:
        shifts.append(jj)
        jj //= 2

    def merge_body(ki, carry):
        k = jnp.int32(2) << ki
        for jj in shifts:
            @pl.when(jj < k)
            def _stage():
                v = key_ref[...]
                ix = idx_ref[...]
                pld = out_ref[0]
                is_hi = (pos & jj) != 0
                dirm = is_hi ^ ((pos & k) != 0)

                def partner(a):
                    return jnp.where(is_hi, pltpu.roll(a, jj, 0),
                                     pltpu.roll(a, n - jj, 0))

                pv = partner(v)
                pix = partner(ix)
                ppl = partner(pld)
                takes = _swap_mask(pv, v, pix, ix) ^ dirm
                key_ref[...] = jnp.where(takes, pv, v)
                idx_ref[...] = jnp.where(takes, pix, ix)
                out_ref[0] = jnp.where(takes, ppl, pld)
        return carry

    n_phases = n.bit_length() - 1
    jax.lax.fori_loop(0, n_phases, merge_body, 0)


def kernel(start_patch_token, cls_token, W_f, b_f, W_s, b_s, Temperature):
    B, N, D = start_patch_token.shape
    S = W_s.shape[0]
    out = pl.pallas_call(
        _token_select_kernel,
        grid=(B,),
        in_specs=[
            pl.BlockSpec((1, N, D), lambda b: (b, 0, 0)),
            pl.BlockSpec((1, 1, D), lambda b: (b, 0, 0)),
            pl.BlockSpec(W_f.shape, lambda b: (0, 0)),
            pl.BlockSpec((1, W_f.shape[0]), lambda b: (0, 0)),
            pl.BlockSpec(W_s.shape, lambda b: (0, 0)),
            pl.BlockSpec((1, S), lambda b: (0, 0)),
        ],
        out_specs=pl.BlockSpec((1, N, D), lambda b: (b, 0, 0)),
        out_shape=jax.ShapeDtypeStruct((B, N, D), jnp.float32),
        scratch_shapes=[
            pltpu.VMEM((N, S), jnp.float32),
            pltpu.VMEM((N, S), jnp.int32),
        ],
        compiler_params=pltpu.CompilerParams(
            dimension_semantics=("arbitrary",)),
    )(start_patch_token, cls_token.reshape(B, 1, D), W_f,
      b_f.reshape(1, -1), W_s, b_s.reshape(1, -1))
    return out


# speedup vs baseline: 7.3185x; 1.1954x over previous
import jax
import jax.numpy as jnp
from jax.experimental import pallas as pl
from jax.experimental.pallas import tpu as pltpu


def _swap_mask(vb, va, ib, ia):
    return (vb > va) | ((vb == va) & (ib < ia))


def _stage_big(v, ix, pld, j, k, n):
    m = n // (2 * j)
    s = v.shape[-1]

    def split(a):
        r = a.reshape(m, 2, j, s)
        return r[:, 0], r[:, 1]

    va, vb = split(v)
    ia, ib = split(ix)
    pa, pb = split(pld)
    swap = _swap_mask(vb, va, ib, ia)
    if k < n:
        o = jax.lax.broadcasted_iota(jnp.int32, (m, 1, 1), 0)
        down = ((o * (2 * j)) & k) != 0
        swap = swap ^ down

    def merge(a, b):
        na = jnp.where(swap, b, a)
        nb = jnp.where(swap, a, b)
        return jnp.concatenate([na[:, None], nb[:, None]], axis=1).reshape(n, s)

    return merge(va, vb), merge(ia, ib), merge(pa, pb)


def _stage_small(v, ix, pld, j, k, n, pos):
    is_hi = (pos & j) != 0
    dirm = is_hi ^ ((pos & k) != 0)

    def partner(a):
        return jnp.where(is_hi, pltpu.roll(a, j, 0), pltpu.roll(a, n - j, 0))

    pv = partner(v)
    pix = partner(ix)
    ppl = partner(pld)
    takes = _swap_mask(pv, v, pix, ix) ^ dirm
    return (jnp.where(takes, pv, v), jnp.where(takes, pix, ix),
            jnp.where(takes, ppl, pld))


def _token_select_kernel(x_ref, cls_ref, wf_ref, bf_ref, ws_ref, bs_ref,
                         out_ref, key_ref, idx_ref):
    x = x_ref[0]
    n = x.shape[0]

    half = jax.lax.dot_general(x, wf_ref[...], (((1,), (1,)), ((), ())),
                               preferred_element_type=jnp.float32)
    half = jnp.maximum(half + bf_ref[...], 0.0)
    cls_half = jax.lax.dot_general(cls_ref[0], wf_ref[...],
                                   (((1,), (1,)), ((), ())),
                                   preferred_element_type=jnp.float32)
    cls_half = cls_half + bf_ref[...]
    cat = jnp.concatenate(
        [jnp.broadcast_to(cls_half, half.shape), half], axis=1)
    scores = jax.lax.dot_general(cat, ws_ref[...], (((1,), (1,)), ((), ())),
                                 preferred_element_type=jnp.float32)
    scores = jnp.maximum(scores + bs_ref[...], 0.0)

    pos = jax.lax.broadcasted_iota(jnp.int32, scores.shape, 0)
    key_ref[...] = scores
    idx_ref[...] = pos
    out_ref[0] = x

    shifts = []
    jj = n // 2
    while jj >= 1:
        shifts.append(jj)
        jj //= 2

    s = scores.shape[-1]

    def merge_body(ki, carry):
        k = jnp.int32(2) << ki
        for jj in shifts:
            if jj >= 8:
                @pl.when(jj < k)
                def _stage_aligned():
                    m = n // (2 * jj)

                    def split(a):
                        r = a.reshape(m, 2, jj, s)
                        return r[:, 0], r[:, 1]

                    va, vb = split(key_ref[...])
                    ia, ib = split(idx_ref[...])
                    pa, pb = split(out_ref[0])
                    o = jax.lax.broadcasted_iota(jnp.int32, (m, 1, 1), 0)
                    down = ((o * (2 * jj)) & k) != 0
                    swap = _swap_mask(vb, va, ib, ia) ^ down

                    def merge(a, b):
                        na = jnp.where(swap, b, a)
                        nb = jnp.where(swap, a, b)
                        return jnp.concatenate(
                            [na[:, None], nb[:, None]], axis=1).reshape(n, s)

                    key_ref[...] = merge(va, vb)
                    idx_ref[...] = merge(ia, ib)
                    out_ref[0] = merge(pa, pb)
            else:
                @pl.when(jj < k)
                def _stage_roll():
                    v = key_ref[...]
                    ix = idx_ref[...]
                    pld = out_ref[0]
                    is_hi = (pos & jj) != 0
                    dirm = is_hi ^ ((pos & k) != 0)

                    def partner(a):
                        return jnp.where(is_hi, pltpu.roll(a, jj, 0),
                                         pltpu.roll(a, n - jj, 0))

                    pv = partner(v)
                    pix = partner(ix)
                    ppl = partner(pld)
                    takes = _swap_mask(pv, v, pix, ix) ^ dirm
                    key_ref[...] = jnp.where(takes, pv, v)
                    idx_ref[...] = jnp.where(takes, pix, ix)
                    out_ref[0] = jnp.where(takes, ppl, pld)
        return carry

    n_phases = n.bit_length() - 1
    jax.lax.fori_loop(0, n_phases, merge_body, 0)


def kernel(start_patch_token, cls_token, W_f, b_f, W_s, b_s, Temperature):
    B, N, D = start_patch_token.shape
    S = W_s.shape[0]
    out = pl.pallas_call(
        _token_select_kernel,
        grid=(B,),
        in_specs=[
            pl.BlockSpec((1, N, D), lambda b: (b, 0, 0)),
            pl.BlockSpec((1, 1, D), lambda b: (b, 0, 0)),
            pl.BlockSpec(W_f.shape, lambda b: (0, 0)),
            pl.BlockSpec((1, W_f.shape[0]), lambda b: (0, 0)),
            pl.BlockSpec(W_s.shape, lambda b: (0, 0)),
            pl.BlockSpec((1, S), lambda b: (0, 0)),
        ],
        out_specs=pl.BlockSpec((1, N, D), lambda b: (b, 0, 0)),
        out_shape=jax.ShapeDtypeStruct((B, N, D), jnp.float32),
        scratch_shapes=[
            pltpu.VMEM((N, S), jnp.float32),
            pltpu.VMEM((N, S), jnp.int32),
        ],
        compiler_params=pltpu.CompilerParams(
            dimension_semantics=("arbitrary",)),
    )(start_patch_token, cls_token.reshape(B, 1, D), W_f,
      b_f.reshape(1, -1), W_s, b_s.reshape(1, -1))
    return out


# speedup vs baseline: 7.7856x; 1.0638x over previous
import jax
import jax.numpy as jnp
from jax.experimental import pallas as pl
from jax.experimental.pallas import tpu as pltpu


def _swap_mask(vb, va, ib, ia):
    return (vb > va) | ((vb == va) & (ib < ia))


def _stage_big(v, ix, pld, j, k, n):
    m = n // (2 * j)
    s = v.shape[-1]

    def split(a):
        r = a.reshape(m, 2, j, s)
        return r[:, 0], r[:, 1]

    va, vb = split(v)
    ia, ib = split(ix)
    pa, pb = split(pld)
    swap = _swap_mask(vb, va, ib, ia)
    if k < n:
        o = jax.lax.broadcasted_iota(jnp.int32, (m, 1, 1), 0)
        down = ((o * (2 * j)) & k) != 0
        swap = swap ^ down

    def merge(a, b):
        na = jnp.where(swap, b, a)
        nb = jnp.where(swap, a, b)
        return jnp.concatenate([na[:, None], nb[:, None]], axis=1).reshape(n, s)

    return merge(va, vb), merge(ia, ib), merge(pa, pb)


def _stage_small(v, ix, pld, j, k, n, pos):
    is_hi = (pos & j) != 0
    dirm = is_hi ^ ((pos & k) != 0)

    def partner(a):
        return jnp.where(is_hi, pltpu.roll(a, j, 0), pltpu.roll(a, n - j, 0))

    pv = partner(v)
    pix = partner(ix)
    ppl = partner(pld)
    takes = _swap_mask(pv, v, pix, ix) ^ dirm
    return (jnp.where(takes, pv, v), jnp.where(takes, pix, ix),
            jnp.where(takes, ppl, pld))


def _token_select_kernel(x_ref, cls_ref, wf_ref, bf_ref, ws_ref, bs_ref,
                         out_ref, key_ref, idx_ref):
    x = x_ref[0]
    n = x.shape[0]

    half = jax.lax.dot_general(x, wf_ref[...], (((1,), (1,)), ((), ())),
                               preferred_element_type=jnp.float32)
    half = jnp.maximum(half + bf_ref[...], 0.0)
    cls_half = jax.lax.dot_general(cls_ref[0], wf_ref[...],
                                   (((1,), (1,)), ((), ())),
                                   preferred_element_type=jnp.float32)
    cls_half = cls_half + bf_ref[...]
    cat = jnp.concatenate(
        [jnp.broadcast_to(cls_half, half.shape), half], axis=1)
    scores = jax.lax.dot_general(cat, ws_ref[...], (((1,), (1,)), ((), ())),
                                 preferred_element_type=jnp.float32)
    scores = jnp.maximum(scores + bs_ref[...], 0.0)

    pos = jax.lax.broadcasted_iota(jnp.int32, scores.shape, 0)
    key_ref[...] = scores
    idx_ref[...] = pos
    out_ref[0] = x

    shifts = []
    jj = n // 2
    while jj >= 1:
        shifts.append(jj)
        jj //= 2

    s = scores.shape[-1]

    def merge_body(ki, carry):
        k = jnp.int32(2) << ki
        for jj in shifts:
            if jj >= 8:
                @pl.when(jj < k)
                def _stage_aligned():
                    m = n // (2 * jj)

                    def split(a):
                        r = a.reshape(m, 2, jj, s)
                        return r[:, 0], r[:, 1]

                    va, vb = split(key_ref[...])
                    ia, ib = split(idx_ref[...])
                    pa, pb = split(out_ref[0])
                    o = jax.lax.broadcasted_iota(jnp.int32, (m, 1, 1), 0)
                    down = ((o * (2 * jj)) & k) != 0
                    swap = _swap_mask(vb, va, ib, ia) ^ down

                    def merge(a, b):
                        na = jnp.where(swap, b, a)
                        nb = jnp.where(swap, a, b)
                        return jnp.concatenate(
                            [na[:, None], nb[:, None]], axis=1).reshape(n, s)

                    key_ref[...] = merge(va, vb)
                    idx_ref[...] = merge(ia, ib)
                    out_ref[0] = merge(pa, pb)
        def tail(stage_shifts):
            def run():
                v = key_ref[...]
                ix = idx_ref[...]
                pld = out_ref[0]
                for jj in stage_shifts:
                    is_hi = (pos & jj) != 0
                    dirm = is_hi ^ ((pos & k) != 0)

                    def partner(a, jj=jj, is_hi=is_hi):
                        return jnp.where(is_hi, pltpu.roll(a, jj, 0),
                                         pltpu.roll(a, n - jj, 0))

                    pv = partner(v)
                    pix = partner(ix)
                    ppl = partner(pld)
                    takes = _swap_mask(pv, v, pix, ix) ^ dirm
                    v = jnp.where(takes, pv, v)
                    ix = jnp.where(takes, pix, ix)
                    pld = jnp.where(takes, ppl, pld)
                key_ref[...] = v
                idx_ref[...] = ix
                out_ref[0] = pld
            return run

        pl.when(k >= 8)(tail((4, 2, 1)))
        pl.when(k == 4)(tail((2, 1)))
        pl.when(k == 2)(tail((1,)))
        return carry

    n_phases = n.bit_length() - 1
    jax.lax.fori_loop(0, n_phases, merge_body, 0)


def kernel(start_patch_token, cls_token, W_f, b_f, W_s, b_s, Temperature):
    B, N, D = start_patch_token.shape
    S = W_s.shape[0]
    out = pl.pallas_call(
        _token_select_kernel,
        grid=(B,),
        in_specs=[
            pl.BlockSpec((1, N, D), lambda b: (b, 0, 0)),
            pl.BlockSpec((1, 1, D), lambda b: (b, 0, 0)),
            pl.BlockSpec(W_f.shape, lambda b: (0, 0)),
            pl.BlockSpec((1, W_f.shape[0]), lambda b: (0, 0)),
            pl.BlockSpec(W_s.shape, lambda b: (0, 0)),
            pl.BlockSpec((1, S), lambda b: (0, 0)),
        ],
        out_specs=pl.BlockSpec((1, N, D), lambda b: (b, 0, 0)),
        out_shape=jax.ShapeDtypeStruct((B, N, D), jnp.float32),
        scratch_shapes=[
            pltpu.VMEM((N, S), jnp.float32),
            pltpu.VMEM((N, S), jnp.int32),
        ],
        compiler_params=pltpu.CompilerParams(
            dimension_semantics=("arbitrary",)),
    )(start_patch_token, cls_token.reshape(B, 1, D), W_f,
      b_f.reshape(1, -1), W_s, b_s.reshape(1, -1))
    return out
